# baseline (speedup 1.0000x reference)
import jax
import jax.numpy as jnp
from jax import lax
from jax.experimental import pallas as pl
from jax.experimental.pallas import tpu as pltpu
from jax.experimental.pallas import tpu_sc as plsc

_L = 16385
_D = 64
_NREL = 1000
_CHUNK = 128
_NW = 32
_CPW = 4
_ROWS_PW = _CHUNK * _CPW
_PAD = _NW * _ROWS_PW + _CHUNK


def _sc_body(path_hbm, table_hbm, out_hbm, pbuf, cidx, rows, sem_g, sem_w):
    nc = 2
    wid = lax.axis_index("s") * nc + lax.axis_index("c")
    off = (lax.iota(jnp.int32, 16) & 1) * _NREL

    base = wid * _ROWS_PW
    pltpu.sync_copy(path_hbm.at[pl.ds(base, _ROWS_PW)], pbuf)
    for j in range(_CPW):
        cj = cidx.at[j]
        for k in range(_CHUNK // 16):
            cj[pl.ds(k * 16, 16)] = pbuf[pl.ds(j * _CHUNK + k * 16, 16)] + off
    gathers = [
        pltpu.async_copy(
            table_hbm.at[cidx.at[j]],
            rows.at[pl.ds(j * _CHUNK, _CHUNK)],
            sem_g,
        )
        for j in range(_CPW)
    ]
    writes = []
    for j in range(_CPW):
        gathers[j].wait()
        writes.append(
            pltpu.async_copy(
                rows.at[pl.ds(j * _CHUNK, _CHUNK)],
                out_hbm.at[pl.ds(base + j * _CHUNK, _CHUNK)],
                sem_w,
            )
        )

    @pl.when(wid == 0)
    def _():
        tbase = _NW * _ROWS_PW
        pltpu.sync_copy(path_hbm.at[pl.ds(tbase, _CHUNK)],
                        pbuf.at[pl.ds(0, _CHUNK)])
        cj = cidx.at[0]
        for k in range(_CHUNK // 16):
            s = pl.ds(k * 16, 16)
            cj[s] = pbuf[s] + off
        pltpu.async_copy(
            table_hbm.at[cidx.at[0]],
            rows.at[pl.ds(0, _CHUNK)],
            sem_g,
        ).wait()
        pltpu.async_copy(
            rows.at[pl.ds(0, _CHUNK)],
            out_hbm.at[pl.ds(tbase, _CHUNK)],
            sem_w,
        ).wait()

    for w in writes:
        w.wait()


def _tc_relayout(x_ref, o_ref, e_ref, o1_ref):
    @pl.when(pl.program_id(0) == 0)
    def _():
        r = lax.broadcasted_iota(jnp.int32, (256, 512), 0)
        c = lax.broadcasted_iota(jnp.int32, (256, 512), 1)
        e_ref[...] = (c == 2 * r).astype(jnp.float32)
        o1_ref[...] = (c == 2 * r + 1).astype(jnp.float32)

    x = x_ref[...]
    dn = (((0,), (0,)), ((), ()))
    o_ref[...] = lax.dot_general(
        x[:, :_D], e_ref[...], dn, precision=lax.Precision.HIGHEST
    ) + lax.dot_general(
        x[:, _D:], o1_ref[...], dn, precision=lax.Precision.HIGHEST
    )


def kernel(path, W_ent, W_rel):
    table = jnp.concatenate([W_ent[:_NREL], W_rel[:_NREL]], axis=0)
    p = jnp.zeros((_PAD,), jnp.int32).at[:_L].set(path.astype(jnp.int32))
    mesh = plsc.VectorSubcoreMesh(core_axis_name="c", subcore_axis_name="s")
    sc_out = pl.kernel(
        _sc_body,
        mesh=mesh,
        compiler_params=pltpu.CompilerParams(use_tc_tiling_on_sc=False),
        out_type=jax.ShapeDtypeStruct((_PAD, _D), jnp.float32),
        scratch_types=[
            pltpu.VMEM((_ROWS_PW,), jnp.int32),
            pltpu.VMEM((_CPW, _CHUNK), jnp.int32),
            pltpu.VMEM((_ROWS_PW, _D), jnp.float32),
            pltpu.SemaphoreType.DMA,
            pltpu.SemaphoreType.DMA,
        ],
    )(p, table)

    x = sc_out.reshape(_PAD // 2, 2 * _D)
    yt = pl.pallas_call(
        _tc_relayout,
        grid=(33,),
        in_specs=[pl.BlockSpec((256, 2 * _D), lambda b: (b, 0))],
        out_specs=pl.BlockSpec((_D, 512), lambda b: (0, b)),
        out_shape=jax.ShapeDtypeStruct((_D, _L), jnp.float32),
        scratch_shapes=[
            pltpu.VMEM((256, 512), jnp.float32),
            pltpu.VMEM((256, 512), jnp.float32),
        ],
    )(x)
    return yt.T


# speedup vs baseline: 1.0319x; 1.0319x over previous
import jax
import jax.numpy as jnp
from jax import lax
from jax.experimental import pallas as pl
from jax.experimental.pallas import tpu as pltpu
from jax.experimental.pallas import tpu_sc as plsc

_L = 16385
_D = 64
_NREL = 1000
_CHUNK = 128
_NW = 32
_CPW = 4
_ROWS_PW = _CHUNK * _CPW
_NCH = _NW * _CPW + 1
_PAD = _NCH * _CHUNK


def _sc_body(path_hbm, table_hbm, out_hbm, pbuf, cidx, rows, sem_g, sem_w):
    nc = 2
    wid = lax.axis_index("s") * nc + lax.axis_index("c")
    off = (lax.iota(jnp.int32, 16) & 1) * _NREL

    base = wid * _ROWS_PW
    pltpu.sync_copy(path_hbm.at[pl.ds(base, _ROWS_PW)], pbuf)
    for j in range(_CPW):
        cj = cidx.at[j]
        for k in range(_CHUNK // 16):
            cj[pl.ds(k * 16, 16)] = pbuf[pl.ds(j * _CHUNK + k * 16, 16)] + off
    gathers = [
        pltpu.async_copy(
            table_hbm.at[cidx.at[j]],
            rows.at[pl.ds(j * _CHUNK, _CHUNK)],
            sem_g,
        )
        for j in range(_CPW)
    ]
    writes = []
    for j in range(_CPW):
        gathers[j].wait()
        writes.append(
            pltpu.async_copy(
                rows.at[pl.ds(j * _CHUNK, _CHUNK)],
                out_hbm.at[pl.ds(base + j * _CHUNK, _CHUNK)],
                sem_w,
            )
        )

    @pl.when(wid == 0)
    def _():
        tbase = _NW * _ROWS_PW
        pltpu.sync_copy(path_hbm.at[pl.ds(tbase, _CHUNK)],
                        pbuf.at[pl.ds(0, _CHUNK)])
        cj = cidx.at[0]
        for k in range(_CHUNK // 16):
            s = pl.ds(k * 16, 16)
            cj[s] = pbuf[s] + off
        pltpu.async_copy(
            table_hbm.at[cidx.at[0]],
            rows.at[pl.ds(0, _CHUNK)],
            sem_g,
        ).wait()
        pltpu.async_copy(
            rows.at[pl.ds(0, _CHUNK)],
            out_hbm.at[pl.ds(tbase, _CHUNK)],
            sem_w,
        ).wait()

    for w in writes:
        w.wait()


def _tc_unpack(x_ref, o_ref):
    x = x_ref[...]
    o_ref[pl.Slice(0, 256, 2), :] = x[:, :_D]
    o_ref[pl.Slice(1, 256, 2), :] = x[:, _D:]


def kernel(path, W_ent, W_rel):
    table = jnp.concatenate([W_ent[:_NREL], W_rel[:_NREL]], axis=0)
    p = jnp.zeros((_PAD,), jnp.int32).at[:_L].set(path.astype(jnp.int32))
    mesh = plsc.VectorSubcoreMesh(core_axis_name="c", subcore_axis_name="s")
    sc_out = pl.kernel(
        _sc_body,
        mesh=mesh,
        compiler_params=pltpu.CompilerParams(use_tc_tiling_on_sc=False),
        out_type=jax.ShapeDtypeStruct((_PAD, _D), jnp.float32),
        scratch_types=[
            pltpu.VMEM((_ROWS_PW,), jnp.int32),
            pltpu.VMEM((_CPW, _CHUNK), jnp.int32),
            pltpu.VMEM((_ROWS_PW, _D), jnp.float32),
            pltpu.SemaphoreType.DMA,
            pltpu.SemaphoreType.DMA,
        ],
    )(p, table)

    x = sc_out.reshape(_PAD // 2, 2 * _D)
    out = pl.pallas_call(
        _tc_unpack,
        grid=(33,),
        in_specs=[pl.BlockSpec((256, 2 * _D), lambda b: (b, 0))],
        out_specs=pl.BlockSpec((512, _D), lambda b: (b, 0)),
        out_shape=jax.ShapeDtypeStruct((_L, _D), jnp.float32),
    )(x)
    return out


# speedup vs baseline: 1.3796x; 1.3370x over previous
import jax
import jax.numpy as jnp
from jax import lax
from jax.experimental import pallas as pl
from jax.experimental.pallas import tpu as pltpu
from jax.experimental.pallas import tpu_sc as plsc

_L = 16385
_D = 64
_NREL = 1000
_CHUNK = 128
_NW = 32
_CPW = 4
_ROWS_PW = _CHUNK * _CPW
_MAIN = _NW * _ROWS_PW
_PPAD = _MAIN + 16


def _sc_body(path_hbm, table_hbm, out_hbm, pbuf, cidx, rows, tidx, trows,
             sem_g, sem_w):
    nc = 2
    wid = lax.axis_index("s") * nc + lax.axis_index("c")
    off = (lax.iota(jnp.int32, 16) & 1) * _NREL

    base = wid * _ROWS_PW
    pltpu.sync_copy(path_hbm.at[pl.ds(base, _ROWS_PW)], pbuf)
    for j in range(_CPW):
        cj = cidx.at[j]
        for k in range(_CHUNK // 16):
            cj[pl.ds(k * 16, 16)] = pbuf[pl.ds(j * _CHUNK + k * 16, 16)] + off
    gathers = [
        pltpu.async_copy(
            table_hbm.at[cidx.at[j]],
            rows.at[pl.ds(j * _CHUNK, _CHUNK)],
            sem_g,
        )
        for j in range(_CPW)
    ]
    writes = []
    for j in range(_CPW):
        gathers[j].wait()
        writes.append(
            pltpu.async_copy(
                rows.at[pl.ds(j * _CHUNK, _CHUNK)],
                out_hbm.at[pl.ds(base + j * _CHUNK, _CHUNK)],
                sem_w,
            )
        )

    @pl.when(wid == 0)
    def _():
        pltpu.sync_copy(path_hbm.at[pl.ds(_MAIN, 16)], tidx)
        tidx[...] = tidx[...] + off
        pltpu.async_copy(table_hbm.at[tidx], trows, sem_g).wait()
        pltpu.async_copy(
            trows.at[pl.ds(0, 1)], out_hbm.at[pl.ds(_MAIN, 1)], sem_w
        ).wait()

    for w in writes:
        w.wait()


def kernel(path, W_ent, W_rel):
    table = jnp.concatenate([W_ent[:_NREL], W_rel[:_NREL]], axis=0)
    p = jnp.zeros((_PPAD,), jnp.int32).at[:_L].set(path.astype(jnp.int32))
    mesh = plsc.VectorSubcoreMesh(core_axis_name="c", subcore_axis_name="s")
    out = pl.kernel(
        _sc_body,
        mesh=mesh,
        compiler_params=pltpu.CompilerParams(use_tc_tiling_on_sc=False),
        out_type=jax.ShapeDtypeStruct((_L, _D), jnp.float32),
        scratch_types=[
            pltpu.VMEM((_ROWS_PW,), jnp.int32),
            pltpu.VMEM((_CPW, _CHUNK), jnp.int32),
            pltpu.VMEM((_ROWS_PW, _D), jnp.float32),
            pltpu.VMEM((16,), jnp.int32),
            pltpu.VMEM((16, _D), jnp.float32),
            pltpu.SemaphoreType.DMA,
            pltpu.SemaphoreType.DMA,
        ],
    )(p, table)
    return out


# speedup vs baseline: 1.4258x; 1.0335x over previous
import jax
import jax.numpy as jnp
from jax import lax
from jax.experimental import pallas as pl
from jax.experimental.pallas import tpu as pltpu
from jax.experimental.pallas import tpu_sc as plsc

_L = 16385
_D = 64
_NREL = 1000
_CHUNK = 128
_NW = 32
_CPW = 4
_ROWS_PW = _CHUNK * _CPW
_MAIN = _NW * _ROWS_PW
_PPAD = _MAIN + 16


def _sc_body(path_hbm, table_hbm, out_hbm, pbuf, cidx, rows, tidx, trows,
             sem_g, sem_w):
    nc = 2
    wid = lax.axis_index("s") * nc + lax.axis_index("c")
    off = (lax.iota(jnp.int32, 16) & 1) * _NREL

    base = wid * _ROWS_PW
    pltpu.sync_copy(path_hbm.at[pl.ds(base, _ROWS_PW)], pbuf)
    for j in range(_CPW):
        cj = cidx.at[j]
        for k in range(_CHUNK // 16):
            cj[pl.ds(k * 16, 16)] = pbuf[pl.ds(j * _CHUNK + k * 16, 16)] + off
    gathers = [
        pltpu.async_copy(
            table_hbm.at[cidx.at[j]],
            rows.at[pl.ds(j * _CHUNK, _CHUNK)],
            sem_g,
        )
        for j in range(_CPW)
    ]
    writes = []
    for j in range(_CPW):
        gathers[j].wait()
        writes.append(
            pltpu.async_copy(
                rows.at[pl.ds(j * _CHUNK, _CHUNK)],
                out_hbm.at[pl.ds(base + j * _CHUNK, _CHUNK)],
                sem_w,
            )
        )

    @pl.when(wid == 0)
    def _():
        pltpu.sync_copy(path_hbm.at[pl.ds(_MAIN - 16, 16)], tidx)
        pltpu.sync_copy(path_hbm.at[pl.ds(_MAIN, 1)], tidx.at[pl.ds(0, 1)])
        tidx[...] = tidx[...] + off
        pltpu.async_copy(table_hbm.at[tidx], trows, sem_g).wait()
        pltpu.async_copy(
            trows.at[pl.ds(0, 1)], out_hbm.at[pl.ds(_MAIN, 1)], sem_w
        ).wait()

    for w in writes:
        w.wait()


def kernel(path, W_ent, W_rel):
    table = jnp.concatenate([W_ent[:_NREL], W_rel[:_NREL]], axis=0)
    p = path.astype(jnp.int32)
    mesh = plsc.VectorSubcoreMesh(core_axis_name="c", subcore_axis_name="s")
    out = pl.kernel(
        _sc_body,
        mesh=mesh,
        compiler_params=pltpu.CompilerParams(use_tc_tiling_on_sc=False),
        out_type=jax.ShapeDtypeStruct((_L, _D), jnp.float32),
        scratch_types=[
            pltpu.VMEM((_ROWS_PW,), jnp.int32),
            pltpu.VMEM((_CPW, _CHUNK), jnp.int32),
            pltpu.VMEM((_ROWS_PW, _D), jnp.float32),
            pltpu.VMEM((16,), jnp.int32),
            pltpu.VMEM((16, _D), jnp.float32),
            pltpu.SemaphoreType.DMA,
            pltpu.SemaphoreType.DMA,
        ],
    )(p, table)
    return out


# speedup vs baseline: 1.4260x; 1.0002x over previous
import jax
import jax.numpy as jnp
from jax import lax
from jax.experimental import pallas as pl
from jax.experimental.pallas import tpu as pltpu
from jax.experimental.pallas import tpu_sc as plsc

_L = 16385
_D = 64
_NREL = 1000
_CHUNK = 128
_NW = 32
_CPW = 4
_ROWS_PW = _CHUNK * _CPW
_MAIN = _NW * _ROWS_PW
_PPAD = _MAIN + 16


def _sc_body(path_hbm, table_hbm, out_hbm, pbuf, cidx, rows, tidx, trows,
             sem_g, sem_w):
    nc = 2
    wid = lax.axis_index("s") * nc + lax.axis_index("c")
    off = (lax.iota(jnp.int32, 16) & 1) * _NREL

    base = wid * _ROWS_PW
    pltpu.sync_copy(path_hbm.at[pl.ds(base, _ROWS_PW)], pbuf)
    gathers = []
    for j in range(_CPW):
        cj = cidx.at[j]
        for k in range(_CHUNK // 16):
            cj[pl.ds(k * 16, 16)] = pbuf[pl.ds(j * _CHUNK + k * 16, 16)] + off
        gathers.append(
            pltpu.async_copy(
                table_hbm.at[cj],
                rows.at[pl.ds(j * _CHUNK, _CHUNK)],
                sem_g,
            )
        )
    writes = []
    for j in range(_CPW):
        gathers[j].wait()
        writes.append(
            pltpu.async_copy(
                rows.at[pl.ds(j * _CHUNK, _CHUNK)],
                out_hbm.at[pl.ds(base + j * _CHUNK, _CHUNK)],
                sem_w,
            )
        )

    @pl.when(wid == 0)
    def _():
        pltpu.sync_copy(path_hbm.at[pl.ds(_MAIN - 16, 16)], tidx)
        pltpu.sync_copy(path_hbm.at[pl.ds(_MAIN, 1)], tidx.at[pl.ds(0, 1)])
        tidx[...] = tidx[...] + off
        pltpu.async_copy(table_hbm.at[tidx], trows, sem_g).wait()
        pltpu.async_copy(
            trows.at[pl.ds(0, 1)], out_hbm.at[pl.ds(_MAIN, 1)], sem_w
        ).wait()

    for w in writes:
        w.wait()


def kernel(path, W_ent, W_rel):
    table = jnp.concatenate([W_ent[:_NREL], W_rel[:_NREL]], axis=0)
    p = path.astype(jnp.int32)
    mesh = plsc.VectorSubcoreMesh(core_axis_name="c", subcore_axis_name="s")
    out = pl.kernel(
        _sc_body,
        mesh=mesh,
        compiler_params=pltpu.CompilerParams(use_tc_tiling_on_sc=False),
        out_type=jax.ShapeDtypeStruct((_L, _D), jnp.float32),
        scratch_types=[
            pltpu.VMEM((_ROWS_PW,), jnp.int32),
            pltpu.VMEM((_CPW, _CHUNK), jnp.int32),
            pltpu.VMEM((_ROWS_PW, _D), jnp.float32),
            pltpu.VMEM((16,), jnp.int32),
            pltpu.VMEM((16, _D), jnp.float32),
            pltpu.SemaphoreType.DMA,
            pltpu.SemaphoreType.DMA,
        ],
    )(p, table)
    return out
